# baseline (speedup 1.0000x reference)
import functools

import jax
import jax.numpy as jnp
from jax import lax
from jax.experimental import pallas as pl
from jax.experimental.pallas import tpu as pltpu
from jax.experimental.pallas import tpu_sc as plsc

_LANES = 16
_NBUF = 2
_CHUNK_ROWS = 32


def _sc_encoder(batch, num_patches, proj_dim, dtype):
    info = plsc.get_sparse_core_info()
    n_workers = info.num_cores * info.num_subcores
    rows_per_w = num_patches // n_workers
    halves = rows_per_w // _CHUNK_ROWS
    n_chunks = batch * halves

    mesh = plsc.VectorSubcoreMesh(core_axis_name="c", subcore_axis_name="s")

    @functools.partial(
        pl.kernel,
        mesh=mesh,
        out_type=jax.ShapeDtypeStruct((batch, num_patches, proj_dim), dtype),
        scratch_types=[
            pltpu.VMEM((rows_per_w, proj_dim), dtype),
            [pltpu.VMEM((_CHUNK_ROWS, proj_dim), dtype) for _ in range(_NBUF)],
            [pltpu.VMEM((_CHUNK_ROWS, proj_dim), dtype) for _ in range(_NBUF)],
            [pltpu.SemaphoreType.DMA for _ in range(_NBUF)],
            [pltpu.SemaphoreType.DMA for _ in range(_NBUF)],
        ],
    )
    def k(patch_hbm, pos_hbm, out_hbm, pos_v, in_v, out_v, in_sem, out_sem):
        wid = lax.axis_index("s") * info.num_cores + lax.axis_index("c")
        base = wid * rows_per_w
        pltpu.sync_copy(pos_hbm.at[pl.ds(base, rows_per_w)], pos_v)

        def chunk_rows(c):
            return pl.ds(base + (c % halves) * _CHUNK_ROWS, _CHUNK_ROWS)

        for s in range(_NBUF):
            pltpu.async_copy(
                patch_hbm.at[s // halves, chunk_rows(s)], in_v[s], in_sem[s]
            )

        def per_quad(q, carry):
            for s in range(_NBUF):
                c = q * _NBUF + s
                b = q * (_NBUF // halves) + s // halves
                rows = chunk_rows(s)
                pltpu.make_async_copy(
                    patch_hbm.at[b, rows], in_v[s], in_sem[s]
                ).wait()

                @pl.when(c >= _NBUF)
                def _():
                    pltpu.make_async_copy(
                        out_v[s], out_hbm.at[b, rows], out_sem[s]
                    ).wait()

                pos_off = (s % halves) * _CHUNK_ROWS

                @plsc.parallel_loop(0, _CHUNK_ROWS)
                def _(i):
                    for j in range(proj_dim // _LANES):
                        sl = pl.ds(j * _LANES, _LANES)
                        out_v[s][i, sl] = in_v[s][i, sl] + pos_v[pos_off + i, sl]

                pltpu.async_copy(out_v[s], out_hbm.at[b, rows], out_sem[s])

                @pl.when(c + _NBUF < n_chunks)
                def _():
                    nb = b + _NBUF // halves
                    pltpu.async_copy(
                        patch_hbm.at[nb, rows], in_v[s], in_sem[s]
                    )

            return carry

        lax.fori_loop(0, n_chunks // _NBUF, per_quad, 0, unroll=False)

        for s in range(_NBUF):
            c = n_chunks - _NBUF + s
            pltpu.make_async_copy(
                out_v[s],
                out_hbm.at[c // halves, chunk_rows(s)],
                out_sem[s],
            ).wait()

    return k


def kernel(patch, pos_table):
    batch, num_patches, proj_dim = patch.shape
    return _sc_encoder(batch, num_patches, proj_dim, patch.dtype)(
        patch, pos_table
    )
